# baseline (speedup 1.0000x reference)
import jax
import jax.numpy as jnp
from jax.experimental import pallas as pl
from jax.experimental.pallas import tpu as pltpu

_BLOCK = 2000


def _ogc_matmul_kernel(x_ref, w_ref, o_ref):
    o_ref[...] = jax.lax.dot_general(
        x_ref[...],
        w_ref[...],
        dimension_numbers=(((1,), (1,)), ((), ())),
        preferred_element_type=jnp.float32,
    )


def kernel(x, W):
    n, nfeat = x.shape
    nclass = W.shape[0]
    grid = n // _BLOCK
    return pl.pallas_call(
        _ogc_matmul_kernel,
        grid=(grid,),
        in_specs=[
            pl.BlockSpec((_BLOCK, nfeat), lambda i: (i, 0)),
            pl.BlockSpec((nclass, nfeat), lambda i: (0, 0)),
        ],
        out_specs=pl.BlockSpec((_BLOCK, nclass), lambda i: (i, 0)),
        out_shape=jax.ShapeDtypeStruct((n, nclass), jnp.float32),
        compiler_params=pltpu.CompilerParams(
            dimension_semantics=("parallel",),
        ),
    )(x, W)


# speedup vs baseline: 1.9717x; 1.9717x over previous
import jax
import jax.numpy as jnp
from jax.experimental import pallas as pl
from jax.experimental.pallas import tpu as pltpu

_BLOCK = 2048


def _ogc_matmul_kernel(x_ref, w_ref, o_ref):
    o_ref[...] = jax.lax.dot_general(
        w_ref[...],
        x_ref[...],
        dimension_numbers=(((1,), (1,)), ((), ())),
        preferred_element_type=jnp.float32,
    )


def kernel(x, W):
    n, nfeat = x.shape
    nclass = W.shape[0]
    grid = pl.cdiv(n, _BLOCK)
    out_t = pl.pallas_call(
        _ogc_matmul_kernel,
        grid=(grid,),
        in_specs=[
            pl.BlockSpec((_BLOCK, nfeat), lambda i: (i, 0)),
            pl.BlockSpec((nclass, nfeat), lambda i: (0, 0)),
        ],
        out_specs=pl.BlockSpec((nclass, _BLOCK), lambda i: (0, i)),
        out_shape=jax.ShapeDtypeStruct((nclass, n), jnp.float32),
        compiler_params=pltpu.CompilerParams(
            dimension_semantics=("arbitrary",),
        ),
    )(x, W)
    return out_t.T


# speedup vs baseline: 3.0672x; 1.5556x over previous
import jax
import jax.numpy as jnp
from jax.experimental import pallas as pl
from jax.experimental.pallas import tpu as pltpu

_BLOCK = 2048
_DEPTH = 8


def _ogc_matmul_kernel(x_hbm, w_ref, o_ref, buf, sems):
    i = pl.program_id(0)
    nsteps = pl.num_programs(0)
    n = x_hbm.shape[0]

    edge_rows = n - (nsteps - 1) * _BLOCK

    def full_copy(step, slot):
        return pltpu.make_async_copy(
            x_hbm.at[pl.ds(step * _BLOCK, _BLOCK), :],
            buf.at[slot],
            sems.at[slot],
        )

    def edge_copy(slot):
        return pltpu.make_async_copy(
            x_hbm.at[pl.ds((nsteps - 1) * _BLOCK, edge_rows), :],
            buf.at[slot, pl.ds(0, edge_rows)],
            sems.at[slot],
        )

    def start_tile(step, slot):
        @pl.when(step < nsteps - 1)
        def _full():
            full_copy(step, slot).start()

        @pl.when(step == nsteps - 1)
        def _edge():
            edge_copy(slot).start()

    @pl.when(i == 0)
    def _warmup():
        for j in range(_DEPTH):
            start_tile(j, j)

    slot = jax.lax.rem(i, _DEPTH)

    @pl.when(i < nsteps - 1)
    def _wait_full():
        full_copy(i, slot).wait()

    @pl.when(i == nsteps - 1)
    def _wait_edge():
        edge_copy(slot).wait()
    o_ref[...] = jax.lax.dot_general(
        w_ref[...],
        buf[slot],
        dimension_numbers=(((1,), (1,)), ((), ())),
        preferred_element_type=jnp.float32,
    )

    nxt = i + _DEPTH

    @pl.when(nxt < nsteps)
    def _prefetch():
        start_tile(nxt, slot)


def kernel(x, W):
    n, nfeat = x.shape
    nclass = W.shape[0]
    grid = pl.cdiv(n, _BLOCK)
    out_t = pl.pallas_call(
        _ogc_matmul_kernel,
        grid=(grid,),
        in_specs=[
            pl.BlockSpec(memory_space=pl.ANY),
            pl.BlockSpec((nclass, nfeat), lambda i: (0, 0)),
        ],
        out_specs=pl.BlockSpec((nclass, _BLOCK), lambda i: (0, i)),
        out_shape=jax.ShapeDtypeStruct((nclass, n), jnp.float32),
        scratch_shapes=[
            pltpu.VMEM((_DEPTH, _BLOCK, nfeat), jnp.float32),
            pltpu.SemaphoreType.DMA((_DEPTH,)),
        ],
        compiler_params=pltpu.CompilerParams(
            dimension_semantics=("arbitrary",),
        ),
    )(x, W)
    return out_t.T


# speedup vs baseline: 3.8886x; 1.2678x over previous
import jax
import jax.numpy as jnp
from jax.experimental import pallas as pl
from jax.experimental.pallas import tpu as pltpu

_BLOCK = 4096
_DEPTH = 8


def _ogc_matmul_kernel(x_hbm, w_ref, o_ref, buf, sems):
    i = pl.program_id(0)
    nsteps = pl.num_programs(0)
    n = x_hbm.shape[0]

    edge_rows = n - (nsteps - 1) * _BLOCK

    def full_copy(step, slot):
        return pltpu.make_async_copy(
            x_hbm.at[pl.ds(step * _BLOCK, _BLOCK), :],
            buf.at[slot],
            sems.at[slot],
        )

    def edge_copy(slot):
        return pltpu.make_async_copy(
            x_hbm.at[pl.ds((nsteps - 1) * _BLOCK, edge_rows), :],
            buf.at[slot, pl.ds(0, edge_rows)],
            sems.at[slot],
        )

    def start_tile(step, slot):
        @pl.when(step < nsteps - 1)
        def _full():
            full_copy(step, slot).start()

        @pl.when(step == nsteps - 1)
        def _edge():
            edge_copy(slot).start()

    @pl.when(i == 0)
    def _warmup():
        for j in range(_DEPTH):
            start_tile(j, j)

    slot = jax.lax.rem(i, _DEPTH)

    @pl.when(i < nsteps - 1)
    def _wait_full():
        full_copy(i, slot).wait()

    @pl.when(i == nsteps - 1)
    def _wait_edge():
        edge_copy(slot).wait()
    o_ref[...] = jax.lax.dot_general(
        w_ref[...],
        buf[slot],
        dimension_numbers=(((1,), (1,)), ((), ())),
        preferred_element_type=jnp.float32,
    )

    nxt = i + _DEPTH

    @pl.when(nxt < nsteps)
    def _prefetch():
        start_tile(nxt, slot)


def kernel(x, W):
    n, nfeat = x.shape
    nclass = W.shape[0]
    grid = pl.cdiv(n, _BLOCK)
    out_t = pl.pallas_call(
        _ogc_matmul_kernel,
        grid=(grid,),
        in_specs=[
            pl.BlockSpec(memory_space=pl.ANY),
            pl.BlockSpec((nclass, nfeat), lambda i: (0, 0)),
        ],
        out_specs=pl.BlockSpec((nclass, _BLOCK), lambda i: (0, i)),
        out_shape=jax.ShapeDtypeStruct((nclass, n), jnp.float32),
        scratch_shapes=[
            pltpu.VMEM((_DEPTH, _BLOCK, nfeat), jnp.float32),
            pltpu.SemaphoreType.DMA((_DEPTH,)),
        ],
        compiler_params=pltpu.CompilerParams(
            dimension_semantics=("arbitrary",),
        ),
    )(x, W)
    return out_t.T
